# baseline (speedup 1.0000x reference)
import functools

import jax
import jax.numpy as jnp
from jax import lax
from jax.experimental import pallas as pl
from jax.experimental.pallas import tpu as pltpu
from jax.experimental.pallas import tpu_sc as plsc

_NUM_CORES = 2
_NUM_SUBCORES = 16
_NUM_WORKERS = _NUM_CORES * _NUM_SUBCORES
_CHUNK = 64
_NSEM = 4


def kernel(x, embedding):
    seq_len = x.shape[1]
    features = embedding.shape[1]
    rows_per_worker = seq_len // _NUM_WORKERS
    n_chunks = rows_per_worker // _CHUNK
    mesh = plsc.VectorSubcoreMesh(core_axis_name="c", subcore_axis_name="s")

    @functools.partial(
        pl.kernel,
        out_type=jax.ShapeDtypeStruct((seq_len, features), embedding.dtype),
        mesh=mesh,
        scratch_types=[pltpu.SemaphoreType.DMA((_NSEM,))],
    )
    def sc_copy(emb_hbm, out_hbm, sems):
        wid = lax.axis_index("s") * _NUM_CORES + lax.axis_index("c")
        base = wid * rows_per_worker

        def copy(j):
            return pltpu.make_async_copy(
                emb_hbm.at[pl.ds(base + j * _CHUNK, _CHUNK)],
                out_hbm.at[pl.ds(base + j * _CHUNK, _CHUNK)],
                sems.at[j % _NSEM],
            )

        for j in range(n_chunks):
            copy(j).start()
        for j in range(n_chunks):
            copy(j).wait()

    return sc_copy(embedding)


# speedup vs baseline: 24.6408x; 24.6408x over previous
import functools

import jax
import jax.numpy as jnp
from jax import lax
from jax.experimental import pallas as pl
from jax.experimental.pallas import tpu as pltpu
from jax.experimental.pallas import tpu_sc as plsc

_NUM_CORES = 2
_NUM_SUBCORES = 16
_NUM_WORKERS = _NUM_CORES * _NUM_SUBCORES
_CHUNK = 32
_NBUF = 2


def kernel(x, embedding):
    seq_len = x.shape[1]
    features = embedding.shape[1]
    rows_per_worker = seq_len // _NUM_WORKERS
    n_chunks = rows_per_worker // _CHUNK
    mesh = plsc.VectorSubcoreMesh(core_axis_name="c", subcore_axis_name="s")

    @functools.partial(
        pl.kernel,
        out_type=jax.ShapeDtypeStruct((seq_len, features), embedding.dtype),
        mesh=mesh,
        scratch_types=[
            pltpu.VMEM_SHARED((_NUM_SUBCORES, _NBUF, _CHUNK, features),
                              jnp.float32),
            pltpu.SemaphoreType.DMA((_NBUF,)),
            pltpu.SemaphoreType.DMA((_NBUF,)),
        ],
    )
    def sc_copy(emb_hbm, out_hbm, shared, in_sems, out_sems):
        cid = lax.axis_index("c")
        sid = lax.axis_index("s")
        wid = sid * _NUM_CORES + cid
        base = wid * rows_per_worker

        def in_copy(j):
            b = j % _NBUF
            return pltpu.make_async_copy(
                emb_hbm.at[pl.ds(base + j * _CHUNK, _CHUNK)],
                shared.at[sid, b],
                in_sems.at[b],
            )

        def out_copy(j):
            b = j % _NBUF
            return pltpu.make_async_copy(
                shared.at[sid, b],
                out_hbm.at[pl.ds(base + j * _CHUNK, _CHUNK)],
                out_sems.at[b],
            )

        in_copy(0).start()
        for j in range(n_chunks):
            in_copy(j).wait()
            out_copy(j).start()
            if j + 1 < n_chunks:
                if j >= 1:
                    out_copy(j - 1).wait()
                in_copy(j + 1).start()
        out_copy(n_chunks - 2).wait()
        out_copy(n_chunks - 1).wait()

    return sc_copy(embedding)


# speedup vs baseline: 24.7199x; 1.0032x over previous
import functools

import jax
import jax.numpy as jnp
from jax import lax
from jax.experimental import pallas as pl
from jax.experimental.pallas import tpu as pltpu
from jax.experimental.pallas import tpu_sc as plsc

_NUM_CORES = 2
_NUM_SUBCORES = 16
_NUM_WORKERS = _NUM_CORES * _NUM_SUBCORES
_CHUNK = 32
_NBUF = 2


def kernel(x, embedding):
    seq_len = x.shape[1]
    features = embedding.shape[1]
    rows_per_worker = seq_len // _NUM_WORKERS
    rows_per_path = rows_per_worker // 2
    n_chunks = rows_per_path // _CHUNK
    mesh = plsc.VectorSubcoreMesh(core_axis_name="c", subcore_axis_name="s")

    @functools.partial(
        pl.kernel,
        out_type=jax.ShapeDtypeStruct((seq_len, features), embedding.dtype),
        mesh=mesh,
        scratch_types=[
            pltpu.VMEM((_NBUF, _CHUNK, features), jnp.float32),
            pltpu.VMEM_SHARED((_NUM_SUBCORES, _NBUF, _CHUNK, features),
                              jnp.float32),
            pltpu.SemaphoreType.DMA((_NBUF,)),
            pltpu.SemaphoreType.DMA((_NBUF,)),
            pltpu.SemaphoreType.DMA((_NBUF,)),
            pltpu.SemaphoreType.DMA((_NBUF,)),
        ],
    )
    def sc_copy(emb_hbm, out_hbm, tbuf, shared, tin_sems, tout_sems,
                sin_sems, sout_sems):
        cid = lax.axis_index("c")
        sid = lax.axis_index("s")
        wid = sid * _NUM_CORES + cid
        base_a = wid * rows_per_worker
        base_b = base_a + rows_per_path

        def a_in(j):
            b = j % _NBUF
            return pltpu.make_async_copy(
                emb_hbm.at[pl.ds(base_a + j * _CHUNK, _CHUNK)],
                tbuf.at[b], tin_sems.at[b])

        def a_out(j):
            b = j % _NBUF
            return pltpu.make_async_copy(
                tbuf.at[b],
                out_hbm.at[pl.ds(base_a + j * _CHUNK, _CHUNK)],
                tout_sems.at[b])

        def b_in(j):
            b = j % _NBUF
            return pltpu.make_async_copy(
                emb_hbm.at[pl.ds(base_b + j * _CHUNK, _CHUNK)],
                shared.at[sid, b], sin_sems.at[b])

        def b_out(j):
            b = j % _NBUF
            return pltpu.make_async_copy(
                shared.at[sid, b],
                out_hbm.at[pl.ds(base_b + j * _CHUNK, _CHUNK)],
                sout_sems.at[b])

        a_in(0).start()
        b_in(0).start()
        for j in range(n_chunks):
            a_in(j).wait()
            a_out(j).start()
            b_in(j).wait()
            b_out(j).start()
            if j + 1 < n_chunks:
                if j >= 1:
                    a_out(j - 1).wait()
                    b_out(j - 1).wait()
                a_in(j + 1).start()
                b_in(j + 1).start()
        for j in (n_chunks - 2, n_chunks - 1):
            a_out(j).wait()
            b_out(j).wait()

    return sc_copy(embedding)


# speedup vs baseline: 24.9600x; 1.0097x over previous
import functools

import jax
import jax.numpy as jnp
from jax import lax
from jax.experimental import pallas as pl
from jax.experimental.pallas import tpu as pltpu
from jax.experimental.pallas import tpu_sc as plsc

_NUM_CORES = 2
_NUM_SUBCORES = 16
_NUM_WORKERS = _NUM_CORES * _NUM_SUBCORES
_CHUNK = 64


def kernel(x, embedding):
    seq_len = x.shape[1]
    features = embedding.shape[1]
    rows_per_worker = seq_len // _NUM_WORKERS
    rows_per_path = rows_per_worker // 2
    n_chunks = rows_per_path // _CHUNK
    mesh = plsc.VectorSubcoreMesh(core_axis_name="c", subcore_axis_name="s")

    @functools.partial(
        pl.kernel,
        out_type=jax.ShapeDtypeStruct((seq_len, features), embedding.dtype),
        mesh=mesh,
        scratch_types=[
            pltpu.VMEM((_CHUNK, features), jnp.float32),
            pltpu.VMEM_SHARED((_NUM_SUBCORES, _CHUNK, features), jnp.float32),
            pltpu.SemaphoreType.DMA,
            pltpu.SemaphoreType.DMA,
            pltpu.SemaphoreType.DMA,
            pltpu.SemaphoreType.DMA,
        ],
    )
    def sc_copy(emb_hbm, out_hbm, tbuf, shared, tin_sem, tout_sem,
                sin_sem, sout_sem):
        cid = lax.axis_index("c")
        sid = lax.axis_index("s")
        wid = sid * _NUM_CORES + cid
        base_a = wid * rows_per_worker
        base_b = base_a + rows_per_path

        def a_in(j):
            return pltpu.make_async_copy(
                emb_hbm.at[pl.ds(base_a + j * _CHUNK, _CHUNK)],
                tbuf, tin_sem)

        def a_out(j):
            return pltpu.make_async_copy(
                tbuf,
                out_hbm.at[pl.ds(base_a + j * _CHUNK, _CHUNK)],
                tout_sem)

        def b_in(j):
            return pltpu.make_async_copy(
                emb_hbm.at[pl.ds(base_b + j * _CHUNK, _CHUNK)],
                shared.at[sid], sin_sem)

        def b_out(j):
            return pltpu.make_async_copy(
                shared.at[sid],
                out_hbm.at[pl.ds(base_b + j * _CHUNK, _CHUNK)],
                sout_sem)

        a_in(0).start()
        b_in(0).start()
        for j in range(n_chunks):
            a_in(j).wait()
            a_out(j).start()
            b_in(j).wait()
            b_out(j).start()
            a_out(j).wait()
            b_out(j).wait()
            if j + 1 < n_chunks:
                a_in(j + 1).start()
                b_in(j + 1).start()

    return sc_copy(embedding)
